# Initial kernel scaffold; baseline (speedup 1.0000x reference)
import jax
import jax.numpy as jnp
from jax.experimental import pallas as pl


def kernel(X, edge_idx, W1a, b1a, W1b, b1b, W2a, b2a, W2b, b2b):
    raise NotImplementedError("write your pallas kernel here")


# speedup vs baseline: 8.5560x; 8.5560x over previous
import functools

import jax
import jax.numpy as jnp
from jax import lax
from jax.experimental import pallas as pl
from jax.experimental.pallas import tpu as pltpu
from jax.experimental.pallas import tpu_sc as plsc

N = 10000
D = 128
E = 320000
NCLS = 16

NC = 2
NS = 16
NW = NC * NS
EPW = E // NW
W = 128
K = -(-EPW // W)
EPW_PAD = K * W
N_PAD = 10240
RPT = N_PAD // NS

_mesh = plsc.VectorSubcoreMesh(core_axis_name="c", subcore_axis_name="s")


@functools.partial(
    pl.kernel,
    out_type=jax.ShapeDtypeStruct((NC, N_PAD, D), jnp.float32),
    mesh=_mesh,
    scratch_types=[
        pltpu.VMEM((K, W), jnp.int32),
        pltpu.VMEM((K, W), jnp.int32),
        pltpu.VMEM((W, D), jnp.float32),
        pltpu.VMEM_SHARED((N_PAD, D), jnp.float32),
        pltpu.SemaphoreType.DMA,
    ],
)
def _aggregate(x_hbm, src_hbm, dst_hbm, out_hbm, src_v, dst_v, rows_v, acc, sem):
    c = lax.axis_index("c")
    s = lax.axis_index("s")
    wid = c * NS + s

    @pl.loop(0, W)
    def _(i):
        @pl.loop(0, D, step=16)
        def _(j):
            rows_v[i, pl.ds(j, 16)] = jnp.zeros((16,), jnp.float32)

    @pl.loop(0, RPT, step=W)
    def _(r):
        pltpu.sync_copy(rows_v, acc.at[pl.ds(s * RPT + r, W)])

    pltpu.sync_copy(src_hbm.at[wid], src_v)
    pltpu.sync_copy(dst_hbm.at[wid], dst_v)
    plsc.subcore_barrier()

    @pl.loop(0, K)
    def _(j):
        pltpu.async_copy(x_hbm.at[src_v.at[j]], rows_v, sem).wait()
        pltpu.sync_copy(rows_v, acc.at[dst_v.at[j]], add=True)

    plsc.subcore_barrier()
    pltpu.sync_copy(acc.at[pl.ds(s * RPT, RPT)], out_hbm.at[c, pl.ds(s * RPT, RPT)])


BLK = 2000


def _mlp1(x, parts, w1, b1, w2, b2):
    def body(x_ref, p_ref, w1_ref, b1_ref, w2_ref, b2_ref, o_ref):
        h = x_ref[...] + p_ref[0] + p_ref[1]
        a = jnp.dot(h, w1_ref[...], preferred_element_type=jnp.float32) + b1_ref[...]
        a = jnp.maximum(a, 0.0)
        o = jnp.dot(a, w2_ref[...], preferred_element_type=jnp.float32) + b2_ref[...]
        o_ref[...] = jnp.maximum(o, 0.0)

    return pl.pallas_call(
        body,
        grid=(N // BLK,),
        in_specs=[
            pl.BlockSpec((BLK, D), lambda i: (i, 0)),
            pl.BlockSpec((NC, BLK, D), lambda i: (0, i, 0)),
            pl.BlockSpec((D, D), lambda i: (0, 0)),
            pl.BlockSpec((1, D), lambda i: (0, 0)),
            pl.BlockSpec((D, D), lambda i: (0, 0)),
            pl.BlockSpec((1, D), lambda i: (0, 0)),
        ],
        out_specs=pl.BlockSpec((BLK, D), lambda i: (i, 0)),
        out_shape=jax.ShapeDtypeStruct((N, D), jnp.float32),
    )(x, parts, w1, b1.reshape(1, D), w2, b2.reshape(1, D))


def _mlp2(h, parts, w1, b1, w2, b2):
    def body(h_ref, p_ref, w1_ref, b1_ref, w2_ref, b2_ref, o_ref):
        g = h_ref[...] + p_ref[0] + p_ref[1]
        a = jnp.dot(g, w1_ref[...], preferred_element_type=jnp.float32) + b1_ref[...]
        a = jnp.maximum(a, 0.0)
        y = jnp.dot(a, w2_ref[...], preferred_element_type=jnp.float32) + b2_ref[...]
        m = jnp.max(y, axis=-1, keepdims=True)
        z = y - m
        o_ref[...] = z - jnp.log(jnp.sum(jnp.exp(z), axis=-1, keepdims=True))

    return pl.pallas_call(
        body,
        grid=(N // BLK,),
        in_specs=[
            pl.BlockSpec((BLK, D), lambda i: (i, 0)),
            pl.BlockSpec((NC, BLK, D), lambda i: (0, i, 0)),
            pl.BlockSpec((D, D), lambda i: (0, 0)),
            pl.BlockSpec((1, D), lambda i: (0, 0)),
            pl.BlockSpec((D, NCLS), lambda i: (0, 0)),
            pl.BlockSpec((1, NCLS), lambda i: (0, 0)),
        ],
        out_specs=pl.BlockSpec((BLK, NCLS), lambda i: (i, 0)),
        out_shape=jax.ShapeDtypeStruct((N, NCLS), jnp.float32),
    )(h, parts, w1, b1.reshape(1, D), w2, b2.reshape(1, NCLS))


def _prep_edges(edge_idx):
    src = edge_idx[0].reshape(NW, EPW)
    dst = edge_idx[1].reshape(NW, EPW)
    pad = EPW_PAD - EPW
    w_ids = jnp.arange(NW, dtype=jnp.int32)[:, None]
    j_ids = jnp.arange(pad, dtype=jnp.int32)[None, :]
    pad_src = (j_ids * 89 + w_ids * 113) % N
    pad_dst = N + (j_ids + w_ids * 7) % (N_PAD - N)
    src = jnp.concatenate([src, pad_src], axis=1).reshape(NW, K, W)
    dst = jnp.concatenate([dst, pad_dst], axis=1).reshape(NW, K, W)
    return src, dst


def kernel(X, edge_idx, W1a, b1a, W1b, b1b, W2a, b2a, W2b, b2b):
    srcw, dstw = _prep_edges(edge_idx)
    p1 = _aggregate(X, srcw, dstw)
    h = _mlp1(X, p1, W1a, b1a, W1b, b1b)
    p2 = _aggregate(h, srcw, dstw)
    return _mlp2(h, p2, W2a, b2a, W2b, b2b)


# speedup vs baseline: 13.1151x; 1.5329x over previous
import functools

import jax
import jax.numpy as jnp
from jax import lax
from jax.experimental import pallas as pl
from jax.experimental.pallas import tpu as pltpu
from jax.experimental.pallas import tpu_sc as plsc

N = 10000
D = 128
E = 320000
NCLS = 16

NC = 2
NS = 16
NW = NC * NS
EPW = E // NW
W = 128
K = 80
EPW_PAD = K * W
NA = 10112
RPT = NA // NS
RPT_MAIN = (RPT // W) * W
RPT_TAIL = RPT - RPT_MAIN
DSHIFT = 14

_mesh = plsc.VectorSubcoreMesh(core_axis_name="c", subcore_axis_name="s")


@functools.partial(
    pl.kernel,
    out_type=jax.ShapeDtypeStruct((NC, NA, D), jnp.float32),
    mesh=_mesh,
    scratch_types=[
        pltpu.VMEM((K, W), jnp.int32),
        pltpu.VMEM((2, W), jnp.int32),
        pltpu.VMEM((2, W), jnp.int32),
        pltpu.VMEM((W, D), jnp.float32),
        pltpu.VMEM((W, D), jnp.float32),
        pltpu.VMEM_SHARED((NA, D), jnp.float32),
        pltpu.SemaphoreType.DMA,
        pltpu.SemaphoreType.DMA,
        pltpu.SemaphoreType.DMA,
    ],
)
def _aggregate(x_hbm, edges_hbm, out_hbm, packed_v, src_v, dst_v, rows_a,
               rows_b, acc, sem_a, sem_b, sem_i):
    c = lax.axis_index("c")
    s = lax.axis_index("s")
    wid = c * NS + s

    pltpu.async_copy(edges_hbm.at[wid], packed_v, sem_i)

    @pl.loop(0, W)
    def _(i):
        @pl.loop(0, D, step=16)
        def _(j):
            rows_a[i, pl.ds(j, 16)] = jnp.zeros((16,), jnp.float32)

    @pl.loop(0, RPT_MAIN, step=W)
    def _(r):
        pltpu.sync_copy(rows_a, acc.at[pl.ds(s * RPT + r, W)])

    pltpu.sync_copy(rows_a.at[pl.ds(0, RPT_TAIL)],
                    acc.at[pl.ds(s * RPT + RPT_MAIN, RPT_TAIL)])

    pltpu.make_async_copy(edges_hbm.at[wid], packed_v, sem_i).wait()
    plsc.subcore_barrier()

    def unpack(w, b):
        @pl.loop(0, W, step=16)
        def _(i):
            v = packed_v[w, pl.ds(i, 16)]
            src_v[b, pl.ds(i, 16)] = v & ((1 << DSHIFT) - 1)
            dst_v[b, pl.ds(i, 16)] = v >> DSHIFT

    unpack(0, 0)
    pltpu.async_copy(x_hbm.at[src_v.at[0]], rows_a, sem_a)

    @pl.loop(0, K - 2, step=2)
    def _(j):
        unpack(j + 1, 1)
        pltpu.async_copy(x_hbm.at[src_v.at[1]], rows_b, sem_b)
        pltpu.make_async_copy(x_hbm.at[src_v.at[0]], rows_a, sem_a).wait()
        pltpu.sync_copy(rows_a, acc.at[dst_v.at[0]], add=True)
        unpack(j + 2, 0)
        pltpu.async_copy(x_hbm.at[src_v.at[0]], rows_a, sem_a)
        pltpu.make_async_copy(x_hbm.at[src_v.at[1]], rows_b, sem_b).wait()
        pltpu.sync_copy(rows_b, acc.at[dst_v.at[1]], add=True)

    unpack(K - 1, 1)
    pltpu.async_copy(x_hbm.at[src_v.at[1]], rows_b, sem_b)
    pltpu.make_async_copy(x_hbm.at[src_v.at[0]], rows_a, sem_a).wait()
    pltpu.sync_copy(rows_a, acc.at[dst_v.at[0]], add=True)
    pltpu.make_async_copy(x_hbm.at[src_v.at[1]], rows_b, sem_b).wait()
    pltpu.sync_copy(rows_b, acc.at[dst_v.at[1]], add=True)

    plsc.subcore_barrier()
    pltpu.sync_copy(acc.at[pl.ds(s * RPT, RPT)], out_hbm.at[c, pl.ds(s * RPT, RPT)])


BLK = 2000


def _mlp1(x, parts, w1, b1, w2, b2):
    def body(x_ref, p_ref, w1_ref, b1_ref, w2_ref, b2_ref, o_ref):
        h = x_ref[...] + p_ref[0] + p_ref[1]
        a = jnp.dot(h, w1_ref[...], preferred_element_type=jnp.float32) + b1_ref[...]
        a = jnp.maximum(a, 0.0)
        o = jnp.dot(a, w2_ref[...], preferred_element_type=jnp.float32) + b2_ref[...]
        o_ref[...] = jnp.maximum(o, 0.0)

    return pl.pallas_call(
        body,
        grid=(N // BLK,),
        in_specs=[
            pl.BlockSpec((BLK, D), lambda i: (i, 0)),
            pl.BlockSpec((NC, BLK, D), lambda i: (0, i, 0)),
            pl.BlockSpec((D, D), lambda i: (0, 0)),
            pl.BlockSpec((1, D), lambda i: (0, 0)),
            pl.BlockSpec((D, D), lambda i: (0, 0)),
            pl.BlockSpec((1, D), lambda i: (0, 0)),
        ],
        out_specs=pl.BlockSpec((BLK, D), lambda i: (i, 0)),
        out_shape=jax.ShapeDtypeStruct((N, D), jnp.float32),
    )(x, parts, w1, b1.reshape(1, D), w2, b2.reshape(1, D))


def _mlp2(h, parts, w1, b1, w2, b2):
    def body(h_ref, p_ref, w1_ref, b1_ref, w2_ref, b2_ref, o_ref):
        g = h_ref[...] + p_ref[0] + p_ref[1]
        a = jnp.dot(g, w1_ref[...], preferred_element_type=jnp.float32) + b1_ref[...]
        a = jnp.maximum(a, 0.0)
        y = jnp.dot(a, w2_ref[...], preferred_element_type=jnp.float32) + b2_ref[...]
        m = jnp.max(y, axis=-1, keepdims=True)
        z = y - m
        o_ref[...] = z - jnp.log(jnp.sum(jnp.exp(z), axis=-1, keepdims=True))

    return pl.pallas_call(
        body,
        grid=(N // BLK,),
        in_specs=[
            pl.BlockSpec((BLK, D), lambda i: (i, 0)),
            pl.BlockSpec((NC, BLK, D), lambda i: (0, i, 0)),
            pl.BlockSpec((D, D), lambda i: (0, 0)),
            pl.BlockSpec((1, D), lambda i: (0, 0)),
            pl.BlockSpec((D, NCLS), lambda i: (0, 0)),
            pl.BlockSpec((1, NCLS), lambda i: (0, 0)),
        ],
        out_specs=pl.BlockSpec((BLK, NCLS), lambda i: (i, 0)),
        out_shape=jax.ShapeDtypeStruct((N, NCLS), jnp.float32),
    )(h, parts, w1, b1.reshape(1, D), w2, b2.reshape(1, NCLS))


def _prep_edges(edge_idx):
    src = edge_idx[0].reshape(NW, EPW)
    dst = edge_idx[1].reshape(NW, EPW)
    pad = EPW_PAD - EPW
    w_ids = jnp.arange(NW, dtype=jnp.int32)[:, None]
    j_ids = jnp.arange(pad, dtype=jnp.int32)[None, :]
    pad_src = (j_ids * 131 + w_ids * 977) % N
    pad_dst = N + (j_ids + w_ids * 7) % (NA - N)
    src = jnp.concatenate([src, pad_src], axis=1)
    dst = jnp.concatenate([dst, pad_dst], axis=1)
    return (src | (dst << DSHIFT)).reshape(NW, K, W)


def kernel(X, edge_idx, W1a, b1a, W1b, b1b, W2a, b2a, W2b, b2b):
    edges = _prep_edges(edge_idx)
    p1 = _aggregate(X, edges)
    h = _mlp1(X, p1, W1a, b1a, W1b, b1b)
    p2 = _aggregate(h, edges)
    return _mlp2(h, p2, W2a, b2a, W2b, b2b)


# speedup vs baseline: 13.3742x; 1.0198x over previous
import functools

import jax
import jax.numpy as jnp
from jax import lax
from jax.experimental import pallas as pl
from jax.experimental.pallas import tpu as pltpu
from jax.experimental.pallas import tpu_sc as plsc

N = 10000
D = 128
E = 320000
NCLS = 16

NC = 2
NS = 16
NW = NC * NS
EPW = E // NW
W = 64
K = 160
EPW_PAD = K * W
NA = 10112
RPT = NA // NS
RPT_MAIN = (RPT // W) * W
RPT_TAIL = RPT - RPT_MAIN
DSHIFT = 14

_mesh = plsc.VectorSubcoreMesh(core_axis_name="c", subcore_axis_name="s")


@functools.partial(
    pl.kernel,
    out_type=jax.ShapeDtypeStruct((NC, NA, D), jnp.float32),
    mesh=_mesh,
    scratch_types=[
        pltpu.VMEM((K // 2, 2 * W), jnp.int32),
        pltpu.VMEM((4, W), jnp.int32),
        pltpu.VMEM((4, W), jnp.int32),
        pltpu.VMEM((W, D), jnp.float32),
        pltpu.VMEM((W, D), jnp.float32),
        pltpu.VMEM((W, D), jnp.float32),
        pltpu.VMEM((W, D), jnp.float32),
        pltpu.VMEM_SHARED((NA, D), jnp.float32),
        pltpu.SemaphoreType.DMA,
        pltpu.SemaphoreType.DMA,
        pltpu.SemaphoreType.DMA,
        pltpu.SemaphoreType.DMA,
        pltpu.SemaphoreType.DMA,
        pltpu.SemaphoreType.DMA,
        pltpu.SemaphoreType.DMA,
        pltpu.SemaphoreType.DMA,
        pltpu.SemaphoreType.DMA,
    ],
)
def _aggregate(x_hbm, edges_hbm, out_hbm, packed_v, src_v, dst_v, rows0, rows1,
               rows2, rows3, acc, sem_i, sg0, sg1, sg2, sg3, ss0, ss1, ss2, ss3):
    c = lax.axis_index("c")
    s = lax.axis_index("s")
    wid = c * NS + s
    rows = (rows0, rows1, rows2, rows3)
    sg = (sg0, sg1, sg2, sg3)
    ss = (ss0, ss1, ss2, ss3)

    pltpu.async_copy(edges_hbm.at[wid], packed_v, sem_i)

    @pl.loop(0, W)
    def _(i):
        @pl.loop(0, D, step=16)
        def _(j):
            rows0[i, pl.ds(j, 16)] = jnp.zeros((16,), jnp.float32)

    @pl.loop(0, RPT_MAIN, step=W)
    def _(r):
        pltpu.sync_copy(rows0, acc.at[pl.ds(s * RPT + r, W)])

    pltpu.sync_copy(rows0.at[pl.ds(0, RPT_TAIL)],
                    acc.at[pl.ds(s * RPT + RPT_MAIN, RPT_TAIL)])

    pltpu.make_async_copy(edges_hbm.at[wid], packed_v, sem_i).wait()
    plsc.subcore_barrier()

    def unpack(w, b):
        r = w // 2
        base = (w % 2) * W

        @pl.loop(0, W, step=16)
        def _(i):
            v = packed_v[r, pl.ds(base + i, 16)]
            src_v[b, pl.ds(i, 16)] = v & ((1 << DSHIFT) - 1)
            dst_v[b, pl.ds(i, 16)] = v >> DSHIFT

    def wait_scatter(b):
        pltpu.make_async_copy(rows[b], acc.at[dst_v.at[b]], ss[b]).wait()

    def wait_gather(b):
        pltpu.make_async_copy(x_hbm.at[src_v.at[b]], rows[b], sg[b]).wait()

    @pl.loop(0, K, step=4)
    def _(j):
        for b in range(4):
            w = j + b
            b2 = (b + 2) % 4

            @pl.when(w >= 4)
            def _():
                wait_scatter(b)

            unpack(w, b)
            pltpu.async_copy(x_hbm.at[src_v.at[b]], rows[b], sg[b])

            @pl.when(w >= 2)
            def _():
                wait_gather(b2)
                pltpu.async_copy(rows[b2], acc.at[dst_v.at[b2]], ss[b2],
                                 add=True)

    wait_scatter(0)
    wait_scatter(1)
    wait_gather(2)
    pltpu.async_copy(rows2, acc.at[dst_v.at[2]], ss2, add=True)
    wait_gather(3)
    pltpu.async_copy(rows3, acc.at[dst_v.at[3]], ss3, add=True)
    wait_scatter(2)
    wait_scatter(3)

    plsc.subcore_barrier()
    pltpu.sync_copy(acc.at[pl.ds(s * RPT, RPT)], out_hbm.at[c, pl.ds(s * RPT, RPT)])


BLK = 2000


def _mlp1(x, parts, w1, b1, w2, b2):
    def body(x_ref, p_ref, w1_ref, b1_ref, w2_ref, b2_ref, o_ref):
        h = x_ref[...] + p_ref[0] + p_ref[1]
        a = jnp.dot(h, w1_ref[...], preferred_element_type=jnp.float32) + b1_ref[...]
        a = jnp.maximum(a, 0.0)
        o = jnp.dot(a, w2_ref[...], preferred_element_type=jnp.float32) + b2_ref[...]
        o_ref[...] = jnp.maximum(o, 0.0)

    return pl.pallas_call(
        body,
        grid=(N // BLK,),
        in_specs=[
            pl.BlockSpec((BLK, D), lambda i: (i, 0)),
            pl.BlockSpec((NC, BLK, D), lambda i: (0, i, 0)),
            pl.BlockSpec((D, D), lambda i: (0, 0)),
            pl.BlockSpec((1, D), lambda i: (0, 0)),
            pl.BlockSpec((D, D), lambda i: (0, 0)),
            pl.BlockSpec((1, D), lambda i: (0, 0)),
        ],
        out_specs=pl.BlockSpec((BLK, D), lambda i: (i, 0)),
        out_shape=jax.ShapeDtypeStruct((N, D), jnp.float32),
    )(x, parts, w1, b1.reshape(1, D), w2, b2.reshape(1, D))


def _mlp2(h, parts, w1, b1, w2, b2):
    def body(h_ref, p_ref, w1_ref, b1_ref, w2_ref, b2_ref, o_ref):
        g = h_ref[...] + p_ref[0] + p_ref[1]
        a = jnp.dot(g, w1_ref[...], preferred_element_type=jnp.float32) + b1_ref[...]
        a = jnp.maximum(a, 0.0)
        y = jnp.dot(a, w2_ref[...], preferred_element_type=jnp.float32) + b2_ref[...]
        m = jnp.max(y, axis=-1, keepdims=True)
        z = y - m
        o_ref[...] = z - jnp.log(jnp.sum(jnp.exp(z), axis=-1, keepdims=True))

    return pl.pallas_call(
        body,
        grid=(N // BLK,),
        in_specs=[
            pl.BlockSpec((BLK, D), lambda i: (i, 0)),
            pl.BlockSpec((NC, BLK, D), lambda i: (0, i, 0)),
            pl.BlockSpec((D, D), lambda i: (0, 0)),
            pl.BlockSpec((1, D), lambda i: (0, 0)),
            pl.BlockSpec((D, NCLS), lambda i: (0, 0)),
            pl.BlockSpec((1, NCLS), lambda i: (0, 0)),
        ],
        out_specs=pl.BlockSpec((BLK, NCLS), lambda i: (i, 0)),
        out_shape=jax.ShapeDtypeStruct((N, NCLS), jnp.float32),
    )(h, parts, w1, b1.reshape(1, D), w2, b2.reshape(1, NCLS))


def _prep_edges(edge_idx):
    src = edge_idx[0].reshape(NW, EPW)
    dst = edge_idx[1].reshape(NW, EPW)
    pad = EPW_PAD - EPW
    w_ids = jnp.arange(NW, dtype=jnp.int32)[:, None]
    j_ids = jnp.arange(pad, dtype=jnp.int32)[None, :]
    pad_src = (j_ids * 131 + w_ids * 977) % N
    pad_dst = N + (j_ids + w_ids * 7) % (NA - N)
    src = jnp.concatenate([src, pad_src], axis=1)
    dst = jnp.concatenate([dst, pad_dst], axis=1)
    return (src | (dst << DSHIFT)).reshape(NW, K // 2, 2 * W)


def kernel(X, edge_idx, W1a, b1a, W1b, b1b, W2a, b2a, W2b, b2b):
    edges = _prep_edges(edge_idx)
    p1 = _aggregate(X, edges)
    h = _mlp1(X, p1, W1a, b1a, W1b, b1b)
    p2 = _aggregate(h, edges)
    return _mlp2(h, p2, W2a, b2a, W2b, b2b)


# speedup vs baseline: 13.4441x; 1.0052x over previous
import functools

import jax
import jax.numpy as jnp
from jax import lax
from jax.experimental import pallas as pl
from jax.experimental.pallas import tpu as pltpu
from jax.experimental.pallas import tpu_sc as plsc

N = 10000
D = 128
E = 320000
NCLS = 16

NC = 2
NS = 16
NW = NC * NS
EPW = E // NW
W = 64
K = 160
EPW_PAD = K * W
NA = 10112
RPT = NA // NS
RPT_MAIN = (RPT // W) * W
RPT_TAIL = RPT - RPT_MAIN
DSHIFT = 14

_mesh = plsc.VectorSubcoreMesh(core_axis_name="c", subcore_axis_name="s")


@functools.partial(
    pl.kernel,
    out_type=jax.ShapeDtypeStruct((NC, NA, D), jnp.float32),
    mesh=_mesh,
    scratch_types=[
        pltpu.VMEM((K // 2, 2 * W), jnp.int32),
        pltpu.VMEM((4, W), jnp.int32),
        pltpu.VMEM((4, W), jnp.int32),
        pltpu.VMEM((W, D), jnp.float32),
        pltpu.VMEM((W, D), jnp.float32),
        pltpu.VMEM((W, D), jnp.float32),
        pltpu.VMEM((W, D), jnp.float32),
        pltpu.VMEM_SHARED((NA, D), jnp.float32),
        pltpu.SemaphoreType.DMA,
        pltpu.SemaphoreType.DMA,
        pltpu.SemaphoreType.DMA,
        pltpu.SemaphoreType.DMA,
        pltpu.SemaphoreType.DMA,
        pltpu.SemaphoreType.DMA,
        pltpu.SemaphoreType.DMA,
        pltpu.SemaphoreType.DMA,
        pltpu.SemaphoreType.DMA,
    ],
)
def _aggregate(x_hbm, edges_hbm, out_hbm, packed_v, src_v, dst_v, rows0, rows1,
               rows2, rows3, acc, sem_i, sg0, sg1, sg2, sg3, ss0, ss1, ss2, ss3):
    c = lax.axis_index("c")
    s = lax.axis_index("s")
    wid = c * NS + s
    rows = (rows0, rows1, rows2, rows3)
    sg = (sg0, sg1, sg2, sg3)
    ss = (ss0, ss1, ss2, ss3)

    pltpu.async_copy(edges_hbm.at[wid], packed_v, sem_i)

    @pl.loop(0, W)
    def _(i):
        @pl.loop(0, D, step=16)
        def _(j):
            rows0[i, pl.ds(j, 16)] = jnp.zeros((16,), jnp.float32)

    @pl.loop(0, RPT_MAIN, step=W)
    def _(r):
        pltpu.async_copy(rows0, acc.at[pl.ds(s * RPT + r, W)], sg0)

    pltpu.async_copy(rows0.at[pl.ds(0, RPT_TAIL)],
                     acc.at[pl.ds(s * RPT + RPT_MAIN, RPT_TAIL)], sg0)

    @pl.loop(0, RPT_MAIN, step=W)
    def _(r):
        pltpu.make_async_copy(rows0, acc.at[pl.ds(s * RPT + r, W)], sg0).wait()

    pltpu.make_async_copy(rows0.at[pl.ds(0, RPT_TAIL)],
                          acc.at[pl.ds(s * RPT + RPT_MAIN, RPT_TAIL)],
                          sg0).wait()

    pltpu.make_async_copy(edges_hbm.at[wid], packed_v, sem_i).wait()
    plsc.subcore_barrier()

    def unpack(w, b):
        r = w // 2
        base = (w % 2) * W

        @pl.loop(0, W, step=16)
        def _(i):
            v = packed_v[r, pl.ds(base + i, 16)]
            src_v[b, pl.ds(i, 16)] = v & ((1 << DSHIFT) - 1)
            dst_v[b, pl.ds(i, 16)] = v >> DSHIFT

    def wait_scatter(b):
        pltpu.make_async_copy(rows[b], acc.at[dst_v.at[b]], ss[b]).wait()

    def wait_gather(b):
        pltpu.make_async_copy(x_hbm.at[src_v.at[b]], rows[b], sg[b]).wait()

    @pl.loop(0, K, step=4)
    def _(j):
        for b in range(4):
            w = j + b
            b2 = (b + 2) % 4

            @pl.when(w >= 4)
            def _():
                wait_scatter(b)

            unpack(w, b)
            pltpu.async_copy(x_hbm.at[src_v.at[b]], rows[b], sg[b])

            @pl.when(w >= 2)
            def _():
                wait_gather(b2)
                pltpu.async_copy(rows[b2], acc.at[dst_v.at[b2]], ss[b2],
                                 add=True)

    wait_scatter(0)
    wait_scatter(1)
    wait_gather(2)
    pltpu.async_copy(rows2, acc.at[dst_v.at[2]], ss2, add=True)
    wait_gather(3)
    pltpu.async_copy(rows3, acc.at[dst_v.at[3]], ss3, add=True)
    wait_scatter(2)
    wait_scatter(3)

    plsc.subcore_barrier()
    pltpu.sync_copy(acc.at[pl.ds(s * RPT, RPT)], out_hbm.at[c, pl.ds(s * RPT, RPT)])


BLK = 2000


def _premul(x, w, b):
    def body(x_ref, w_ref, b_ref, o_ref):
        o_ref[...] = (
            jnp.dot(x_ref[...], w_ref[...], preferred_element_type=jnp.float32)
            + b_ref[...]
        )

    return pl.pallas_call(
        body,
        grid=(N // BLK,),
        in_specs=[
            pl.BlockSpec((BLK, D), lambda i: (i, 0)),
            pl.BlockSpec((D, D), lambda i: (0, 0)),
            pl.BlockSpec((1, D), lambda i: (0, 0)),
        ],
        out_specs=pl.BlockSpec((BLK, D), lambda i: (i, 0)),
        out_shape=jax.ShapeDtypeStruct((N, D), jnp.float32),
    )(x, w, b.reshape(1, D))


def _mlp1(u1, parts, w1, w2, b2):
    def body(u_ref, p_ref, w1_ref, w2_ref, b2_ref, o_ref):
        p = p_ref[0] + p_ref[1]
        a = u_ref[...] + jnp.dot(p, w1_ref[...],
                                 preferred_element_type=jnp.float32)
        a = jnp.maximum(a, 0.0)
        o = jnp.dot(a, w2_ref[...], preferred_element_type=jnp.float32) + b2_ref[...]
        o_ref[...] = jnp.maximum(o, 0.0)

    return pl.pallas_call(
        body,
        grid=(N // BLK,),
        in_specs=[
            pl.BlockSpec((BLK, D), lambda i: (i, 0)),
            pl.BlockSpec((NC, BLK, D), lambda i: (0, i, 0)),
            pl.BlockSpec((D, D), lambda i: (0, 0)),
            pl.BlockSpec((D, D), lambda i: (0, 0)),
            pl.BlockSpec((1, D), lambda i: (0, 0)),
        ],
        out_specs=pl.BlockSpec((BLK, D), lambda i: (i, 0)),
        out_shape=jax.ShapeDtypeStruct((N, D), jnp.float32),
    )(u1, parts, w1, w2, b2.reshape(1, D))


def _mlp2(u2, parts, w1, w2, b2):
    def body(u_ref, p_ref, w1_ref, w2_ref, b2_ref, o_ref):
        p = p_ref[0] + p_ref[1]
        a = u_ref[...] + jnp.dot(p, w1_ref[...],
                                 preferred_element_type=jnp.float32)
        a = jnp.maximum(a, 0.0)
        y = jnp.dot(a, w2_ref[...], preferred_element_type=jnp.float32) + b2_ref[...]
        m = jnp.max(y, axis=-1, keepdims=True)
        z = y - m
        o_ref[...] = z - jnp.log(jnp.sum(jnp.exp(z), axis=-1, keepdims=True))

    return pl.pallas_call(
        body,
        grid=(N // BLK,),
        in_specs=[
            pl.BlockSpec((BLK, D), lambda i: (i, 0)),
            pl.BlockSpec((NC, BLK, D), lambda i: (0, i, 0)),
            pl.BlockSpec((D, D), lambda i: (0, 0)),
            pl.BlockSpec((D, NCLS), lambda i: (0, 0)),
            pl.BlockSpec((1, NCLS), lambda i: (0, 0)),
        ],
        out_specs=pl.BlockSpec((BLK, NCLS), lambda i: (i, 0)),
        out_shape=jax.ShapeDtypeStruct((N, NCLS), jnp.float32),
    )(u2, parts, w1, w2, b2.reshape(1, NCLS))


def _prep_edges(edge_idx):
    src = edge_idx[0].reshape(NW, EPW)
    dst = edge_idx[1].reshape(NW, EPW)
    pad = EPW_PAD - EPW
    w_ids = jnp.arange(NW, dtype=jnp.int32)[:, None]
    j_ids = jnp.arange(pad, dtype=jnp.int32)[None, :]
    pad_src = (j_ids * 131 + w_ids * 977) % N
    pad_dst = N + (j_ids + w_ids * 7) % (NA - N)
    src = jnp.concatenate([src, pad_src], axis=1)
    dst = jnp.concatenate([dst, pad_dst], axis=1)
    return (src | (dst << DSHIFT)).reshape(NW, K // 2, 2 * W)


def kernel(X, edge_idx, W1a, b1a, W1b, b1b, W2a, b2a, W2b, b2b):
    edges = _prep_edges(edge_idx)
    u1 = _premul(X, W1a, b1a)
    p1 = _aggregate(X, edges)
    h = _mlp1(u1, p1, W1a, W1b, b1b)
    u2 = _premul(h, W2a, b2a)
    p2 = _aggregate(h, edges)
    return _mlp2(u2, p2, W2a, W2b, b2b)


# speedup vs baseline: 13.4797x; 1.0026x over previous
import functools

import jax
import jax.numpy as jnp
from jax import lax
from jax.experimental import pallas as pl
from jax.experimental.pallas import tpu as pltpu
from jax.experimental.pallas import tpu_sc as plsc

N = 10000
D = 128
E = 320000
NCLS = 16

NC = 2
NS = 16
NW = NC * NS
EPW = E // NW
W = 64
K = 160
EPW_PAD = K * W
NA = 10112
RPT = NA // NS
RPT_MAIN = (RPT // W) * W
RPT_TAIL = RPT - RPT_MAIN
DSHIFT = 14

_mesh = plsc.VectorSubcoreMesh(core_axis_name="c", subcore_axis_name="s")


@functools.partial(
    pl.kernel,
    out_type=jax.ShapeDtypeStruct((NC, NA, D), jnp.float32),
    mesh=_mesh,
    scratch_types=[
        pltpu.VMEM((K // 2, 2 * W), jnp.int32),
        pltpu.VMEM((4, W), jnp.int32),
        pltpu.VMEM((4, W), jnp.int32),
        pltpu.VMEM((W, D), jnp.float32),
        pltpu.VMEM((W, D), jnp.float32),
        pltpu.VMEM((W, D), jnp.float32),
        pltpu.VMEM((W, D), jnp.float32),
        pltpu.VMEM_SHARED((NA, D), jnp.float32),
        pltpu.SemaphoreType.DMA,
        pltpu.SemaphoreType.DMA,
        pltpu.SemaphoreType.DMA,
        pltpu.SemaphoreType.DMA,
        pltpu.SemaphoreType.DMA,
        pltpu.SemaphoreType.DMA,
        pltpu.SemaphoreType.DMA,
        pltpu.SemaphoreType.DMA,
        pltpu.SemaphoreType.DMA,
    ],
)
def _aggregate(x_hbm, edges_hbm, out_hbm, packed_v, src_v, dst_v, rows0, rows1,
               rows2, rows3, acc, sem_i, sg0, sg1, sg2, sg3, ss0, ss1, ss2, ss3):
    c = lax.axis_index("c")
    s = lax.axis_index("s")
    wid = c * NS + s
    rows = (rows0, rows1, rows2, rows3)
    sg = (sg0, sg1, sg2, sg3)
    ss = (ss0, ss1, ss2, ss3)

    pltpu.async_copy(edges_hbm.at[wid], packed_v, sem_i)

    @pl.loop(0, W)
    def _(i):
        @pl.loop(0, D, step=16)
        def _(j):
            rows0[i, pl.ds(j, 16)] = jnp.zeros((16,), jnp.float32)

    @pl.loop(0, RPT_MAIN, step=W)
    def _(r):
        pltpu.async_copy(rows0, acc.at[pl.ds(s * RPT + r, W)], sg0)

    pltpu.async_copy(rows0.at[pl.ds(0, RPT_TAIL)],
                     acc.at[pl.ds(s * RPT + RPT_MAIN, RPT_TAIL)], sg0)

    @pl.loop(0, RPT_MAIN, step=W)
    def _(r):
        pltpu.make_async_copy(rows0, acc.at[pl.ds(s * RPT + r, W)], sg0).wait()

    pltpu.make_async_copy(rows0.at[pl.ds(0, RPT_TAIL)],
                          acc.at[pl.ds(s * RPT + RPT_MAIN, RPT_TAIL)],
                          sg0).wait()

    pltpu.make_async_copy(edges_hbm.at[wid], packed_v, sem_i).wait()
    plsc.subcore_barrier()

    def unpack(w, b):
        r = w // 2
        base = (w % 2) * W

        @pl.loop(0, W, step=16)
        def _(i):
            v = packed_v[r, pl.ds(base + i, 16)]
            src_v[b, pl.ds(i, 16)] = v & ((1 << DSHIFT) - 1)
            dst_v[b, pl.ds(i, 16)] = v >> DSHIFT

    def wait_scatter(b):
        pltpu.make_async_copy(rows[b], acc.at[dst_v.at[b]], ss[b]).wait()

    def wait_gather(b):
        pltpu.make_async_copy(x_hbm.at[src_v.at[b]], rows[b], sg[b]).wait()

    @pl.loop(0, K, step=4)
    def _(j):
        for b in range(4):
            w = j + b
            b2 = (b + 2) % 4

            @pl.when(w >= 4)
            def _():
                wait_scatter(b)

            unpack(w, b)
            pltpu.async_copy(x_hbm.at[src_v.at[b]], rows[b], sg[b])

            @pl.when(w >= 2)
            def _():
                wait_gather(b2)
                pltpu.async_copy(rows[b2], acc.at[dst_v.at[b2]], ss[b2],
                                 add=True)

    wait_scatter(0)
    wait_scatter(1)
    wait_gather(2)
    pltpu.async_copy(rows2, acc.at[dst_v.at[2]], ss2, add=True)
    wait_gather(3)
    pltpu.async_copy(rows3, acc.at[dst_v.at[3]], ss3, add=True)
    wait_scatter(2)
    wait_scatter(3)

    plsc.subcore_barrier()
    pltpu.sync_copy(acc.at[pl.ds(s * RPT, RPT)], out_hbm.at[c, pl.ds(s * RPT, RPT)])


BLK = 2000


def _mlp1(x, parts, w1, b1, w2, b2):
    def body(x_ref, p_ref, w1_ref, b1_ref, w2_ref, b2_ref, o_ref):
        h = x_ref[...] + p_ref[0] + p_ref[1]
        a = jnp.dot(h, w1_ref[...], preferred_element_type=jnp.float32) + b1_ref[...]
        a = jnp.maximum(a, 0.0)
        o = jnp.dot(a, w2_ref[...], preferred_element_type=jnp.float32) + b2_ref[...]
        o_ref[...] = jnp.maximum(o, 0.0)

    return pl.pallas_call(
        body,
        grid=(N // BLK,),
        in_specs=[
            pl.BlockSpec((BLK, D), lambda i: (i, 0)),
            pl.BlockSpec((NC, BLK, D), lambda i: (0, i, 0)),
            pl.BlockSpec((D, D), lambda i: (0, 0)),
            pl.BlockSpec((1, D), lambda i: (0, 0)),
            pl.BlockSpec((D, D), lambda i: (0, 0)),
            pl.BlockSpec((1, D), lambda i: (0, 0)),
        ],
        out_specs=pl.BlockSpec((BLK, D), lambda i: (i, 0)),
        out_shape=jax.ShapeDtypeStruct((N, D), jnp.float32),
    )(x, parts, w1, b1.reshape(1, D), w2, b2.reshape(1, D))


def _mlp2(h, parts, w1, b1, w2, b2):
    def body(h_ref, p_ref, w1_ref, b1_ref, w2_ref, b2_ref, o_ref):
        g = h_ref[...] + p_ref[0] + p_ref[1]
        a = jnp.dot(g, w1_ref[...], preferred_element_type=jnp.float32) + b1_ref[...]
        a = jnp.maximum(a, 0.0)
        y = jnp.dot(a, w2_ref[...], preferred_element_type=jnp.float32) + b2_ref[...]
        m = jnp.max(y, axis=-1, keepdims=True)
        z = y - m
        o_ref[...] = z - jnp.log(jnp.sum(jnp.exp(z), axis=-1, keepdims=True))

    return pl.pallas_call(
        body,
        grid=(N // BLK,),
        in_specs=[
            pl.BlockSpec((BLK, D), lambda i: (i, 0)),
            pl.BlockSpec((NC, BLK, D), lambda i: (0, i, 0)),
            pl.BlockSpec((D, D), lambda i: (0, 0)),
            pl.BlockSpec((1, D), lambda i: (0, 0)),
            pl.BlockSpec((D, NCLS), lambda i: (0, 0)),
            pl.BlockSpec((1, NCLS), lambda i: (0, 0)),
        ],
        out_specs=pl.BlockSpec((BLK, NCLS), lambda i: (i, 0)),
        out_shape=jax.ShapeDtypeStruct((N, NCLS), jnp.float32),
    )(h, parts, w1, b1.reshape(1, D), w2, b2.reshape(1, NCLS))


def _prep_edges(edge_idx):
    src = edge_idx[0].reshape(NW, EPW)
    dst = edge_idx[1].reshape(NW, EPW)
    pad = EPW_PAD - EPW
    w_ids = jnp.arange(NW, dtype=jnp.int32)[:, None]
    j_ids = jnp.arange(pad, dtype=jnp.int32)[None, :]
    pad_src = (j_ids * 131 + w_ids * 977) % N
    pad_dst = N + (j_ids + w_ids * 7) % (NA - N)
    src = jnp.concatenate([src, pad_src], axis=1)
    dst = jnp.concatenate([dst, pad_dst], axis=1)
    return (src | (dst << DSHIFT)).reshape(NW, K // 2, 2 * W)


def kernel(X, edge_idx, W1a, b1a, W1b, b1b, W2a, b2a, W2b, b2b):
    edges = _prep_edges(edge_idx)
    p1 = _aggregate(X, edges)
    h = _mlp1(X, p1, W1a, b1a, W1b, b1b)
    p2 = _aggregate(h, edges)
    return _mlp2(h, p2, W2a, b2a, W2b, b2b)
